# baseline (speedup 1.0000x reference)
import jax
import jax.numpy as jnp
from jax import lax
from jax.experimental import pallas as pl
from jax.experimental.pallas import tpu as pltpu
from jax.experimental.pallas import tpu_sc as plsc

N_NODES = 10000
N_EDGES = 320000
CH = 128
HALF = N_NODES // 2

NC = 2
NS = 16
WIN = 40
EDGES_PER_S = N_EDGES // NS
N_WIN = EDGES_PER_S // WIN
IGNORE = 2147483647

ACC_PER_S = 312
ACC_TAIL = HALF - NS * ACC_PER_S

_MESH = plsc.VectorSubcoreMesh(
    core_axis_name="c", subcore_axis_name="s", num_cores=NC, num_subcores=NS
)


def _sc_spmm_body(y_hbm, r_hbm, co_hbm, ones_hbm, seq_hbm, z_hbm, out_hbm,
                  ridx0_v, cidx0_v, ridx1_v, cidx1_v, sidx_v,
                  rows0_v, rows1_v, gsem0, gsem1, acc_sh):
    c = lax.axis_index("c")
    s = lax.axis_index("s")
    a0 = s * ACC_PER_S

    pltpu.sync_copy(seq_hbm, sidx_v)
    pltpu.sync_copy(ones_hbm.at[sidx_v], rows0_v)
    pltpu.sync_copy(ones_hbm.at[sidx_v], rows1_v)

    pltpu.sync_copy(z_hbm.at[pl.ds(a0, ACC_PER_S)],
                    acc_sh.at[pl.ds(a0, ACC_PER_S)])

    @pl.when(s == 0)
    def _():
        pltpu.sync_copy(z_hbm.at[pl.ds(NS * ACC_PER_S, ACC_TAIL)],
                        acc_sh.at[pl.ds(NS * ACC_PER_S, ACC_TAIL)])

    plsc.subcore_barrier()

    base = c * N_EDGES + s * EDGES_PER_S

    def load_idx(w, ridx_v, cidx_v):
        e0 = base + w * WIN
        pltpu.sync_copy(r_hbm.at[pl.ds(e0, WIN)], ridx_v)
        pltpu.sync_copy(co_hbm.at[pl.ds(e0, WIN)], cidx_v)

    def gather(cidx_v, rows_v, sem):
        return pltpu.async_copy(
            y_hbm.at[plsc.Indices(cidx_v, ignored_value=IGNORE)], rows_v, sem)

    load_idx(0, ridx0_v, cidx0_v)
    gather(cidx0_v, rows0_v, gsem0)
    load_idx(1, ridx1_v, cidx1_v)
    gather(cidx1_v, rows1_v, gsem1)

    def process(w, ridx_v, cidx_v, rows_v, sem):
        pltpu.make_async_copy(
            y_hbm.at[plsc.Indices(cidx_v, ignored_value=IGNORE)], rows_v, sem
        ).wait()
        pltpu.sync_copy(
            rows_v, acc_sh.at[plsc.Indices(ridx_v, ignored_value=IGNORE)],
            add=True)

        @pl.when(w + 2 < N_WIN)
        def _():
            load_idx(w + 2, ridx_v, cidx_v)
            gather(cidx_v, rows_v, sem)

    @pl.loop(0, N_WIN // 2)
    def _(g):
        process(2 * g, ridx0_v, cidx0_v, rows0_v, gsem0)
        process(2 * g + 1, ridx1_v, cidx1_v, rows1_v, gsem1)

    plsc.subcore_barrier()
    o0 = c * HALF + a0
    pltpu.sync_copy(acc_sh.at[pl.ds(a0, ACC_PER_S)],
                    out_hbm.at[pl.ds(o0, ACC_PER_S)])

    @pl.when(s == 0)
    def _():
        pltpu.sync_copy(
            acc_sh.at[pl.ds(NS * ACC_PER_S, ACC_TAIL)],
            out_hbm.at[pl.ds(c * HALF + NS * ACC_PER_S, ACC_TAIL)])


_sc_spmm = pl.kernel(
    _sc_spmm_body,
    out_type=jax.ShapeDtypeStruct((N_NODES, CH), jnp.float32),
    mesh=_MESH,
    scratch_types=[
        pltpu.VMEM((WIN,), jnp.int32),
        pltpu.VMEM((WIN,), jnp.int32),
        pltpu.VMEM((WIN,), jnp.int32),
        pltpu.VMEM((WIN,), jnp.int32),
        pltpu.VMEM((WIN,), jnp.int32),
        pltpu.VMEM((WIN, CH), jnp.float32),
        pltpu.VMEM((WIN, CH), jnp.float32),
        pltpu.SemaphoreType.DMA,
        pltpu.SemaphoreType.DMA,
        pltpu.VMEM_SHARED((HALF, CH), jnp.float32),
    ],
)


_BLK = 1000
_GRID = N_NODES // _BLK
_EROWS = N_EDGES // CH


def _tc_mask_body(row_ref, col_ref, r_ref, c_ref):
    row = row_ref[...]
    col = col_ref[...]
    lo = row < HALF
    ig = jnp.full(row.shape, IGNORE, jnp.int32)
    r_ref[0] = jnp.where(lo, row, ig)
    c_ref[0] = jnp.where(lo, col, ig)
    r_ref[1] = jnp.where(lo, ig, row - HALF)
    c_ref[1] = jnp.where(lo, ig, col)


def _tc_step_body(s_ref, x_ref, w_ref, b_ref, kb_ref, dis_ref, acc_ref,
                  disn_ref, y_ref, accn_ref):
    k = kb_ref[0, 0]
    mode0 = k == 0.0
    s = s_ref[...]
    d0 = s[:, 0:1]
    disc = jnp.where(d0 > 0, lax.rsqrt(jnp.maximum(d0, 1.0)), 0.0)
    dis = jnp.where(mode0, disc, dis_ref[:, 0:1])
    disn_ref[...] = jnp.broadcast_to(dis, (_BLK, 16))
    f = jnp.where(k == 1.0, 1.0, 2.0)
    sub = jnp.where(k == 2.0, 1.0, 0.0)
    t = -(f * (dis * s)) - sub * x_ref[...]
    y_ref[...] = jnp.where(mode0, dis * x_ref[...], dis * t)
    lhs = jnp.where(mode0, x_ref[...], t)
    base = jnp.where(mode0, jnp.broadcast_to(b_ref[...], (_BLK, CH)),
                     acc_ref[...])
    accn_ref[...] = base + jnp.dot(lhs, w_ref[...],
                                   preferred_element_type=jnp.float32,
                                   precision=lax.Precision.HIGHEST)


def _rows_spec(width):
    return pl.BlockSpec((_BLK, width), lambda i: (i, 0))


def _full_spec(shape):
    return pl.BlockSpec(shape, lambda i: (0,) * len(shape))


_tc_step = pl.pallas_call(
    _tc_step_body,
    grid=(_GRID,),
    in_specs=[_rows_spec(CH), _rows_spec(CH), _full_spec((CH, CH)),
              _full_spec((1, CH)), _full_spec((1, CH)),
              _rows_spec(16), _rows_spec(CH)],
    out_specs=[_rows_spec(16), _rows_spec(CH), _rows_spec(CH)],
    out_shape=[
        jax.ShapeDtypeStruct((N_NODES, 16), jnp.float32),
        jax.ShapeDtypeStruct((N_NODES, CH), jnp.float32),
        jax.ShapeDtypeStruct((N_NODES, CH), jnp.float32),
    ],
)

_tc_mask = pl.pallas_call(
    _tc_mask_body,
    grid=(1,),
    in_specs=[_full_spec((_EROWS, CH))] * 2,
    out_specs=[_full_spec((NC, _EROWS, CH))] * 2,
    out_shape=[jax.ShapeDtypeStruct((NC, _EROWS, CH), jnp.int32)] * 2,
)


def kernel(x, edge_index, weight, bias):
    row2d = edge_index[0].reshape(_EROWS, CH)
    col2d = edge_index[1].reshape(_EROWS, CH)
    rcat, ccat = _tc_mask(row2d, col2d)
    rcat = rcat.reshape(NC * N_EDGES)
    ccat = ccat.reshape(NC * N_EDGES)
    ccat_ig = jnp.full((NC * N_EDGES,), IGNORE, jnp.int32)

    zeros128 = jnp.zeros((HALF, CH), jnp.float32)
    ones_win = jnp.ones((WIN, CH), jnp.float32)
    seq_win = jnp.arange(WIN, dtype=jnp.int32)
    bias2d = bias.reshape(1, CH)

    limit = lax.optimization_barrier(jnp.int32(3))

    def cond(carry):
        return carry[0] < limit

    def body(carry):
        k, y, dis16, acc = carry
        col_k = jnp.where(k == 0, ccat_ig, ccat)
        w_k = lax.dynamic_index_in_dim(weight, k, 0, keepdims=False)
        kb = jnp.full((1, CH), 1.0, jnp.float32) * k.astype(jnp.float32)
        sp = _sc_spmm(y, rcat, col_k, ones_win, seq_win, zeros128)
        dis16n, y_next, acc_next = _tc_step(sp, x, w_k, bias2d, kb, dis16, acc)
        return (k + 1, y_next, dis16n, acc_next)

    carry0 = (jnp.int32(0), x, jnp.zeros((N_NODES, 16), jnp.float32),
              jnp.zeros((N_NODES, CH), jnp.float32))
    _, _, _, out = lax.while_loop(cond, body, carry0)
    return out


# speedup vs baseline: 1.2147x; 1.2147x over previous
import functools

import jax
import jax.numpy as jnp
from jax import lax
from jax.experimental import pallas as pl
from jax.experimental.pallas import tpu as pltpu
from jax.experimental.pallas import tpu_sc as plsc

N_NODES = 10000
N_EDGES = 320000
CH = 128

NC = 2
NS = 16
NW = NC * NS
EDGES_PER_W = N_EDGES // NW
WIN = 40
N_WIN = EDGES_PER_W // WIN
IGNORE = 2147483647
ROWS_PER_S = 624
ROWS_TAIL = N_NODES - NS * ROWS_PER_S

_MESH = plsc.VectorSubcoreMesh(
    core_axis_name="c", subcore_axis_name="s", num_cores=NC, num_subcores=NS
)


def _sc_spmm_body(y_hbm, row_hbm, col_hbm, ones_hbm, seq_hbm, z_hbm, out_hbm,
                  ridx_v, cidx_v, sidx_v, rows_v, acc_sh):
    c = lax.axis_index("c")
    s = lax.axis_index("s")
    wid = s * NC + c
    r0 = s * ROWS_PER_S

    pltpu.sync_copy(seq_hbm, sidx_v)
    pltpu.sync_copy(ones_hbm.at[sidx_v], rows_v)

    pltpu.sync_copy(z_hbm.at[pl.ds(r0, ROWS_PER_S)],
                    acc_sh.at[pl.ds(r0, ROWS_PER_S)])

    @pl.when(s == 0)
    def _():
        pltpu.sync_copy(z_hbm.at[pl.ds(NS * ROWS_PER_S, ROWS_TAIL)],
                        acc_sh.at[pl.ds(NS * ROWS_PER_S, ROWS_TAIL)])

    plsc.subcore_barrier()

    base = wid * EDGES_PER_W

    @pl.loop(0, N_WIN)
    def _(w):
        e0 = base + w * WIN
        pltpu.sync_copy(row_hbm.at[pl.ds(e0, WIN)], ridx_v)
        pltpu.sync_copy(col_hbm.at[pl.ds(e0, WIN)], cidx_v)
        pltpu.sync_copy(
            y_hbm.at[plsc.Indices(cidx_v, ignored_value=IGNORE)], rows_v)
        pltpu.sync_copy(rows_v, acc_sh.at[ridx_v], add=True)

    plsc.subcore_barrier()
    o0 = c * N_NODES + r0
    pltpu.sync_copy(acc_sh.at[pl.ds(r0, ROWS_PER_S)],
                    out_hbm.at[pl.ds(o0, ROWS_PER_S)])

    @pl.when(s == 0)
    def _():
        pltpu.sync_copy(
            acc_sh.at[pl.ds(NS * ROWS_PER_S, ROWS_TAIL)],
            out_hbm.at[pl.ds(c * N_NODES + NS * ROWS_PER_S, ROWS_TAIL)])


_sc_spmm = pl.kernel(
    _sc_spmm_body,
    out_type=jax.ShapeDtypeStruct((NC * N_NODES, CH), jnp.float32),
    mesh=_MESH,
    scratch_types=[
        pltpu.VMEM((WIN,), jnp.int32),
        pltpu.VMEM((WIN,), jnp.int32),
        pltpu.VMEM((WIN,), jnp.int32),
        pltpu.VMEM((WIN, CH), jnp.float32),
        pltpu.VMEM_SHARED((N_NODES, CH), jnp.float32),
    ],
)


_BLK = 1000
_GRID = N_NODES // _BLK


def _tc_step_body(sa_ref, sb_ref, x_ref, w_ref, b_ref, kb_ref,
                  dis_ref, acc_ref, disn_ref, y_ref, accn_ref):
    k = kb_ref[0, 0]
    mode0 = k == 0.0
    s = sa_ref[...] + sb_ref[...]
    d0 = s[:, 0:1]
    disc = jnp.where(d0 > 0, lax.rsqrt(jnp.maximum(d0, 1.0)), 0.0)
    dis = jnp.where(mode0, disc, dis_ref[:, 0:1])
    disn_ref[...] = jnp.broadcast_to(dis, (_BLK, 16))
    f = jnp.where(k == 1.0, 1.0, 2.0)
    sub = jnp.where(k == 2.0, 1.0, 0.0)
    t = -(f * (dis * s)) - sub * x_ref[...]
    y_ref[...] = jnp.where(mode0, dis * x_ref[...], dis * t)
    lhs = jnp.where(mode0, x_ref[...], t)
    base = jnp.where(mode0, jnp.broadcast_to(b_ref[...], (_BLK, CH)),
                     acc_ref[...])
    accn_ref[...] = base + jnp.dot(lhs, w_ref[...],
                                   preferred_element_type=jnp.float32,
                                   precision=lax.Precision.HIGHEST)


def _rows_spec(width):
    return pl.BlockSpec((_BLK, width), lambda i: (i, 0))


def _full_spec(shape):
    return pl.BlockSpec(shape, lambda i: (0,) * len(shape))


_tc_step = pl.pallas_call(
    _tc_step_body,
    grid=(_GRID,),
    in_specs=[_rows_spec(CH), _rows_spec(CH), _rows_spec(CH),
              _full_spec((CH, CH)), _full_spec((1, CH)), _full_spec((1, CH)),
              _rows_spec(16), _rows_spec(CH)],
    out_specs=[_rows_spec(16), _rows_spec(CH), _rows_spec(CH)],
    out_shape=[
        jax.ShapeDtypeStruct((N_NODES, 16), jnp.float32),
        jax.ShapeDtypeStruct((N_NODES, CH), jnp.float32),
        jax.ShapeDtypeStruct((N_NODES, CH), jnp.float32),
    ],
)


def kernel(x, edge_index, weight, bias):
    row = edge_index[0]
    col = edge_index[1]
    col_ignore = jnp.full((N_EDGES,), IGNORE, jnp.int32)
    zeros128 = jnp.zeros((N_NODES, CH), jnp.float32)
    ones_win = jnp.ones((WIN, CH), jnp.float32)
    seq_win = jnp.arange(WIN, dtype=jnp.int32)
    bias2d = bias.reshape(1, CH)

    limit = lax.optimization_barrier(jnp.int32(3))

    def cond(carry):
        k = carry[0]
        return k < limit

    def body(carry):
        k, y, dis16, acc = carry
        col_k = jnp.where(k == 0, col_ignore, col)
        w_k = lax.dynamic_index_in_dim(weight, k, 0, keepdims=False)
        kb = jnp.full((1, CH), 1.0, jnp.float32) * k.astype(jnp.float32)
        sp = _sc_spmm(y, row, col_k, ones_win, seq_win, zeros128)
        dis16n, y_next, acc_next = _tc_step(
            sp[:N_NODES], sp[N_NODES:], x, w_k, bias2d, kb, dis16, acc)
        return (k + 1, y_next, dis16n, acc_next)

    carry0 = (jnp.int32(0), x, jnp.zeros((N_NODES, 16), jnp.float32),
              jnp.zeros((N_NODES, CH), jnp.float32))
    _, _, _, out = lax.while_loop(cond, body, carry0)
    return out


# speedup vs baseline: 2.0550x; 1.6918x over previous
import functools

import jax
import jax.numpy as jnp
from jax import lax
from jax.experimental import pallas as pl
from jax.experimental.pallas import tpu as pltpu
from jax.experimental.pallas import tpu_sc as plsc

N_NODES = 10000
N_EDGES = 320000
CH = 128

NC = 2
NS = 16
NW = NC * NS
EDGES_PER_W = N_EDGES // NW
WIN = 40
N_WIN = EDGES_PER_W // WIN
IGNORE = 2147483647
ROWS_PER_S = 624
ROWS_TAIL = N_NODES - NS * ROWS_PER_S

_MESH = plsc.VectorSubcoreMesh(
    core_axis_name="c", subcore_axis_name="s", num_cores=NC, num_subcores=NS
)


def _sc_spmm_body(y_hbm, row_hbm, col_hbm, ones_hbm, seq_hbm, z_hbm, out_hbm,
                  ridx0_v, cidx0_v, ridx1_v, cidx1_v, sidx_v, rows_v,
                  isem0, isem1, acc_sh):
    c = lax.axis_index("c")
    s = lax.axis_index("s")
    wid = s * NC + c
    r0 = s * ROWS_PER_S

    pltpu.sync_copy(seq_hbm, sidx_v)
    pltpu.sync_copy(ones_hbm.at[sidx_v], rows_v)

    pltpu.sync_copy(z_hbm.at[pl.ds(r0, ROWS_PER_S)],
                    acc_sh.at[pl.ds(r0, ROWS_PER_S)])

    @pl.when(s == 0)
    def _():
        pltpu.sync_copy(z_hbm.at[pl.ds(NS * ROWS_PER_S, ROWS_TAIL)],
                        acc_sh.at[pl.ds(NS * ROWS_PER_S, ROWS_TAIL)])

    plsc.subcore_barrier()

    base = wid * EDGES_PER_W
    idx_bufs = ((ridx0_v, cidx0_v, isem0), (ridx1_v, cidx1_v, isem1))

    def start_idx(w, b):
        ridx_v, cidx_v, isem = idx_bufs[b]
        e0 = base + w * WIN
        pltpu.async_copy(row_hbm.at[pl.ds(e0, WIN)], ridx_v, isem)
        pltpu.async_copy(col_hbm.at[pl.ds(e0, WIN)], cidx_v, isem)

    def wait_idx(w, b):
        ridx_v, cidx_v, isem = idx_bufs[b]
        e0 = base + w * WIN
        pltpu.make_async_copy(row_hbm.at[pl.ds(e0, WIN)], ridx_v, isem).wait()
        pltpu.make_async_copy(col_hbm.at[pl.ds(e0, WIN)], cidx_v, isem).wait()

    def process(w, b):
        ridx_v, cidx_v, _ = idx_bufs[b]
        pltpu.sync_copy(
            y_hbm.at[plsc.Indices(cidx_v, ignored_value=IGNORE)], rows_v)
        pltpu.sync_copy(rows_v, acc_sh.at[ridx_v], add=True)

    start_idx(0, 0)
    wait_idx(0, 0)

    @pl.loop(0, N_WIN // 2)
    def _(g):
        w = 2 * g

        @pl.when(w + 1 < N_WIN)
        def _():
            start_idx(w + 1, 1)

        process(w, 0)

        @pl.when(w + 1 < N_WIN)
        def _():
            wait_idx(w + 1, 1)

            @pl.when(w + 2 < N_WIN)
            def _():
                start_idx(w + 2, 0)

            process(w + 1, 1)

            @pl.when(w + 2 < N_WIN)
            def _():
                wait_idx(w + 2, 0)

    plsc.subcore_barrier()
    o0 = c * N_NODES + r0
    pltpu.sync_copy(acc_sh.at[pl.ds(r0, ROWS_PER_S)],
                    out_hbm.at[pl.ds(o0, ROWS_PER_S)])

    @pl.when(s == 0)
    def _():
        pltpu.sync_copy(
            acc_sh.at[pl.ds(NS * ROWS_PER_S, ROWS_TAIL)],
            out_hbm.at[pl.ds(c * N_NODES + NS * ROWS_PER_S, ROWS_TAIL)])


_sc_spmm = pl.kernel(
    _sc_spmm_body,
    out_type=jax.ShapeDtypeStruct((NC * N_NODES, CH), jnp.float32),
    mesh=_MESH,
    scratch_types=[
        pltpu.VMEM((WIN,), jnp.int32),
        pltpu.VMEM((WIN,), jnp.int32),
        pltpu.VMEM((WIN,), jnp.int32),
        pltpu.VMEM((WIN,), jnp.int32),
        pltpu.VMEM((WIN,), jnp.int32),
        pltpu.VMEM((WIN, CH), jnp.float32),
        pltpu.SemaphoreType.DMA,
        pltpu.SemaphoreType.DMA,
        pltpu.VMEM_SHARED((N_NODES, CH), jnp.float32),
    ],
)


_BLK = 1000
_GRID = N_NODES // _BLK


def _tc_step_body(sa_ref, sb_ref, x_ref, w_ref, b_ref, kb_ref,
                  dis_ref, acc_ref, disn_ref, y_ref, accn_ref):
    k = kb_ref[0, 0]
    mode0 = k == 0.0
    s = sa_ref[...] + sb_ref[...]
    d0 = s[:, 0:1]
    disc = jnp.where(d0 > 0, lax.rsqrt(jnp.maximum(d0, 1.0)), 0.0)
    dis = jnp.where(mode0, disc, dis_ref[:, 0:1])
    disn_ref[...] = jnp.broadcast_to(dis, (_BLK, 16))
    f = jnp.where(k == 1.0, 1.0, 2.0)
    sub = jnp.where(k == 2.0, 1.0, 0.0)
    t = -(f * (dis * s)) - sub * x_ref[...]
    y_ref[...] = jnp.where(mode0, dis * x_ref[...], dis * t)
    lhs = jnp.where(mode0, x_ref[...], t)
    base = jnp.where(mode0, jnp.broadcast_to(b_ref[...], (_BLK, CH)),
                     acc_ref[...])
    accn_ref[...] = base + jnp.dot(lhs, w_ref[...],
                                   preferred_element_type=jnp.float32,
                                   precision=lax.Precision.HIGHEST)


def _rows_spec(width):
    return pl.BlockSpec((_BLK, width), lambda i: (i, 0))


def _full_spec(shape):
    return pl.BlockSpec(shape, lambda i: (0,) * len(shape))


_tc_step = pl.pallas_call(
    _tc_step_body,
    grid=(_GRID,),
    in_specs=[_rows_spec(CH), _rows_spec(CH), _rows_spec(CH),
              _full_spec((CH, CH)), _full_spec((1, CH)), _full_spec((1, CH)),
              _rows_spec(16), _rows_spec(CH)],
    out_specs=[_rows_spec(16), _rows_spec(CH), _rows_spec(CH)],
    out_shape=[
        jax.ShapeDtypeStruct((N_NODES, 16), jnp.float32),
        jax.ShapeDtypeStruct((N_NODES, CH), jnp.float32),
        jax.ShapeDtypeStruct((N_NODES, CH), jnp.float32),
    ],
)


def kernel(x, edge_index, weight, bias):
    row = edge_index[0]
    col = edge_index[1]
    col_ignore = jnp.full((N_EDGES,), IGNORE, jnp.int32)
    zeros128 = jnp.zeros((N_NODES, CH), jnp.float32)
    ones_win = jnp.ones((WIN, CH), jnp.float32)
    seq_win = jnp.arange(WIN, dtype=jnp.int32)
    bias2d = bias.reshape(1, CH)

    limit = lax.optimization_barrier(jnp.int32(3))

    def cond(carry):
        k = carry[0]
        return k < limit

    def body(carry):
        k, y, dis16, acc = carry
        col_k = jnp.where(k == 0, col_ignore, col)
        w_k = lax.dynamic_index_in_dim(weight, k, 0, keepdims=False)
        kb = jnp.full((1, CH), 1.0, jnp.float32) * k.astype(jnp.float32)
        sp = _sc_spmm(y, row, col_k, ones_win, seq_win, zeros128)
        dis16n, y_next, acc_next = _tc_step(
            sp[:N_NODES], sp[N_NODES:], x, w_k, bias2d, kb, dis16, acc)
        return (k + 1, y_next, dis16n, acc_next)

    carry0 = (jnp.int32(0), x, jnp.zeros((N_NODES, 16), jnp.float32),
              jnp.zeros((N_NODES, CH), jnp.float32))
    _, _, _, out = lax.while_loop(cond, body, carry0)
    return out


# speedup vs baseline: 2.1192x; 1.0312x over previous
import functools

import jax
import jax.numpy as jnp
from jax import lax
from jax.experimental import pallas as pl
from jax.experimental.pallas import tpu as pltpu
from jax.experimental.pallas import tpu_sc as plsc

N_NODES = 10000
N_EDGES = 320000
CH = 128

NC = 2
NS = 16
NW = NC * NS
EDGES_PER_W = N_EDGES // NW
WIN = 40
N_WIN = EDGES_PER_W // WIN
IGNORE = 2147483647
ROWS_PER_S = 624
ROWS_TAIL = N_NODES - NS * ROWS_PER_S

_MESH = plsc.VectorSubcoreMesh(
    core_axis_name="c", subcore_axis_name="s", num_cores=NC, num_subcores=NS
)


def _sc_spmm_body(y_hbm, row_hbm, col_hbm, ones_hbm, seq_hbm, z_hbm,
                  flag_hbm, out_hbm,
                  ridx0_v, cidx0_v, ridx1_v, cidx1_v, sidx_v, flag_v, rows_v,
                  isem0, isem1, acc_sh):
    c = lax.axis_index("c")
    s = lax.axis_index("s")
    wid = s * NC + c
    r0 = s * ROWS_PER_S

    pltpu.sync_copy(seq_hbm, sidx_v)
    pltpu.sync_copy(ones_hbm.at[sidx_v], rows_v)

    pltpu.sync_copy(z_hbm.at[pl.ds(r0, ROWS_PER_S)],
                    acc_sh.at[pl.ds(r0, ROWS_PER_S)])

    @pl.when(s == 0)
    def _():
        pltpu.sync_copy(z_hbm.at[pl.ds(NS * ROWS_PER_S, ROWS_TAIL)],
                        acc_sh.at[pl.ds(NS * ROWS_PER_S, ROWS_TAIL)])

    pltpu.sync_copy(flag_hbm, flag_v)
    plsc.subcore_barrier()

    is_deg = lax.reduce_max(flag_v[...], axes=(0,)) == 1
    base = wid * EDGES_PER_W
    idx_bufs = ((ridx0_v, cidx0_v, isem0), (ridx1_v, cidx1_v, isem1))

    def start_idx(w, b, with_col):
        ridx_v, cidx_v, isem = idx_bufs[b]
        e0 = base + w * WIN
        pltpu.async_copy(row_hbm.at[pl.ds(e0, WIN)], ridx_v, isem)
        if with_col:
            pltpu.async_copy(col_hbm.at[pl.ds(e0, WIN)], cidx_v, isem)

    def wait_idx(w, b, with_col):
        ridx_v, cidx_v, isem = idx_bufs[b]
        e0 = base + w * WIN
        pltpu.make_async_copy(row_hbm.at[pl.ds(e0, WIN)], ridx_v, isem).wait()
        if with_col:
            pltpu.make_async_copy(
                col_hbm.at[pl.ds(e0, WIN)], cidx_v, isem).wait()

    def window_loop(with_gather):
        def process(w, b):
            ridx_v, cidx_v, _ = idx_bufs[b]
            if with_gather:
                pltpu.sync_copy(y_hbm.at[cidx_v], rows_v)
            pltpu.sync_copy(rows_v, acc_sh.at[ridx_v], add=True)

        start_idx(0, 0, with_gather)
        wait_idx(0, 0, with_gather)

        @pl.loop(0, N_WIN // 2)
        def _(g):
            w = 2 * g
            start_idx(w + 1, 1, with_gather)
            process(w, 0)
            wait_idx(w + 1, 1, with_gather)

            @pl.when(w + 2 < N_WIN)
            def _():
                start_idx(w + 2, 0, with_gather)

            process(w + 1, 1)

            @pl.when(w + 2 < N_WIN)
            def _():
                wait_idx(w + 2, 0, with_gather)

    @pl.when(is_deg)
    def _():
        window_loop(with_gather=False)

    @pl.when(jnp.logical_not(is_deg))
    def _():
        window_loop(with_gather=True)

    plsc.subcore_barrier()
    o0 = c * N_NODES + r0
    pltpu.sync_copy(acc_sh.at[pl.ds(r0, ROWS_PER_S)],
                    out_hbm.at[pl.ds(o0, ROWS_PER_S)])

    @pl.when(s == 0)
    def _():
        pltpu.sync_copy(
            acc_sh.at[pl.ds(NS * ROWS_PER_S, ROWS_TAIL)],
            out_hbm.at[pl.ds(c * N_NODES + NS * ROWS_PER_S, ROWS_TAIL)])


_sc_spmm = pl.kernel(
    _sc_spmm_body,
    out_type=jax.ShapeDtypeStruct((NC * N_NODES, CH), jnp.float32),
    mesh=_MESH,
    compiler_params=pltpu.CompilerParams(needs_layout_passes=False),
    scratch_types=[
        pltpu.VMEM((WIN,), jnp.int32),
        pltpu.VMEM((WIN,), jnp.int32),
        pltpu.VMEM((WIN,), jnp.int32),
        pltpu.VMEM((WIN,), jnp.int32),
        pltpu.VMEM((WIN,), jnp.int32),
        pltpu.VMEM((16,), jnp.int32),
        pltpu.VMEM((WIN, CH), jnp.float32),
        pltpu.SemaphoreType.DMA,
        pltpu.SemaphoreType.DMA,
        pltpu.VMEM_SHARED((N_NODES, CH), jnp.float32),
    ],
)


_BLK = 1000
_GRID = N_NODES // _BLK


def _tc_step_body(sa_ref, sb_ref, x_ref, w_ref, b_ref, kb_ref,
                  dis_ref, acc_ref, disn_ref, y_ref, accn_ref):
    k = kb_ref[0, 0]
    mode0 = k == 0.0
    s = sa_ref[...] + sb_ref[...]
    d0 = s[:, 0:1]
    disc = jnp.where(d0 > 0, lax.rsqrt(jnp.maximum(d0, 1.0)), 0.0)
    dis = jnp.where(mode0, disc, dis_ref[:, 0:1])
    disn_ref[...] = jnp.broadcast_to(dis, (_BLK, 16))
    f = jnp.where(k == 1.0, 1.0, 2.0)
    sub = jnp.where(k == 2.0, 1.0, 0.0)
    t = -(f * (dis * s)) - sub * x_ref[...]
    y_ref[...] = jnp.where(mode0, dis * x_ref[...], dis * t)
    lhs = jnp.where(mode0, x_ref[...], t)
    base = jnp.where(mode0, jnp.broadcast_to(b_ref[...], (_BLK, CH)),
                     acc_ref[...])
    accn_ref[...] = base + jnp.dot(lhs, w_ref[...],
                                   preferred_element_type=jnp.float32,
                                   precision=lax.Precision.HIGHEST)


def _rows_spec(width):
    return pl.BlockSpec((_BLK, width), lambda i: (i, 0))


def _full_spec(shape):
    return pl.BlockSpec(shape, lambda i: (0,) * len(shape))


_tc_step = pl.pallas_call(
    _tc_step_body,
    grid=(_GRID,),
    in_specs=[_rows_spec(CH), _rows_spec(CH), _rows_spec(CH),
              _full_spec((CH, CH)), _full_spec((1, CH)), _full_spec((1, CH)),
              _rows_spec(16), _rows_spec(CH)],
    out_specs=[_rows_spec(16), _rows_spec(CH), _rows_spec(CH)],
    out_shape=[
        jax.ShapeDtypeStruct((N_NODES, 16), jnp.float32),
        jax.ShapeDtypeStruct((N_NODES, CH), jnp.float32),
        jax.ShapeDtypeStruct((N_NODES, CH), jnp.float32),
    ],
)


def kernel(x, edge_index, weight, bias):
    row = edge_index[0]
    col = edge_index[1]
    zeros128 = jnp.zeros((N_NODES, CH), jnp.float32)
    ones_win = jnp.ones((WIN, CH), jnp.float32)
    seq_win = jnp.arange(WIN, dtype=jnp.int32)
    bias2d = bias.reshape(1, CH)

    limit = lax.optimization_barrier(jnp.int32(3))

    def cond(carry):
        k = carry[0]
        return k < limit

    def body(carry):
        k, y, dis16, acc = carry
        flag16 = jnp.full((16,), 1, jnp.int32) * (k == 0).astype(jnp.int32)
        w_k = lax.dynamic_index_in_dim(weight, k, 0, keepdims=False)
        kb = jnp.full((1, CH), 1.0, jnp.float32) * k.astype(jnp.float32)
        sp = _sc_spmm(y, row, col, ones_win, seq_win, zeros128, flag16)
        dis16n, y_next, acc_next = _tc_step(
            sp[:N_NODES], sp[N_NODES:], x, w_k, bias2d, kb, dis16, acc)
        return (k + 1, y_next, dis16n, acc_next)

    carry0 = (jnp.int32(0), x, jnp.zeros((N_NODES, 16), jnp.float32),
              jnp.zeros((N_NODES, CH), jnp.float32))
    _, _, _, out = lax.while_loop(cond, body, carry0)
    return out


# speedup vs baseline: 2.2523x; 1.0628x over previous
import functools

import jax
import jax.numpy as jnp
from jax import lax
from jax.experimental import pallas as pl
from jax.experimental.pallas import tpu as pltpu
from jax.experimental.pallas import tpu_sc as plsc

N_NODES = 10000
N_EDGES = 320000
CH = 128

NC = 2
NS = 16
NW = NC * NS
EDGES_PER_W = N_EDGES // NW
WIN = 40
N_WIN = EDGES_PER_W // WIN
IGNORE = 2147483647
ROWS_PER_S = 624
ROWS_TAIL = N_NODES - NS * ROWS_PER_S

_MESH = plsc.VectorSubcoreMesh(
    core_axis_name="c", subcore_axis_name="s", num_cores=NC, num_subcores=NS
)


def _sc_spmm_body(y_hbm, row_hbm, col_hbm, ones_hbm, seq_hbm, z_hbm,
                  flag_hbm, out_hbm,
                  ridx0_v, cidx0_v, ridx1_v, cidx1_v, sidx_v, flag_v, rows_v,
                  isem0, isem1, ssem0, ssem1, acc_sh):
    c = lax.axis_index("c")
    s = lax.axis_index("s")
    wid = s * NC + c
    r0 = s * ROWS_PER_S

    pltpu.sync_copy(seq_hbm, sidx_v)
    pltpu.sync_copy(ones_hbm.at[sidx_v], rows_v)

    pltpu.sync_copy(z_hbm.at[pl.ds(r0, ROWS_PER_S)],
                    acc_sh.at[pl.ds(r0, ROWS_PER_S)])

    @pl.when(s == 0)
    def _():
        pltpu.sync_copy(z_hbm.at[pl.ds(NS * ROWS_PER_S, ROWS_TAIL)],
                        acc_sh.at[pl.ds(NS * ROWS_PER_S, ROWS_TAIL)])

    pltpu.sync_copy(flag_hbm, flag_v)
    plsc.subcore_barrier()

    is_deg = lax.reduce_max(flag_v[...], axes=(0,)) == 1
    base = wid * EDGES_PER_W
    idx_bufs = ((ridx0_v, cidx0_v, isem0), (ridx1_v, cidx1_v, isem1))

    def start_idx(w, b, with_col):
        ridx_v, cidx_v, isem = idx_bufs[b]
        e0 = base + w * WIN
        pltpu.async_copy(row_hbm.at[pl.ds(e0, WIN)], ridx_v, isem)
        if with_col:
            pltpu.async_copy(col_hbm.at[pl.ds(e0, WIN)], cidx_v, isem)

    def wait_idx(w, b, with_col):
        ridx_v, cidx_v, isem = idx_bufs[b]
        e0 = base + w * WIN
        pltpu.make_async_copy(row_hbm.at[pl.ds(e0, WIN)], ridx_v, isem).wait()
        if with_col:
            pltpu.make_async_copy(
                col_hbm.at[pl.ds(e0, WIN)], cidx_v, isem).wait()

    def window_loop():
        def process(w, b):
            ridx_v, cidx_v, _ = idx_bufs[b]
            pltpu.sync_copy(y_hbm.at[cidx_v], rows_v)
            pltpu.sync_copy(rows_v, acc_sh.at[ridx_v], add=True)

        start_idx(0, 0, True)
        wait_idx(0, 0, True)

        @pl.loop(0, N_WIN // 2)
        def _(g):
            w = 2 * g
            start_idx(w + 1, 1, True)
            process(w, 0)
            wait_idx(w + 1, 1, True)

            @pl.when(w + 2 < N_WIN)
            def _():
                start_idx(w + 2, 0, True)

            process(w + 1, 1)

            @pl.when(w + 2 < N_WIN)
            def _():
                wait_idx(w + 2, 0, True)

    def deg_loop():
        slots = (ridx0_v, ridx1_v, cidx0_v, cidx1_v)
        isems = (isem0, isem1)

        def idx_start(w, b):
            pltpu.async_copy(row_hbm.at[pl.ds(base + w * WIN, WIN)],
                             slots[b], isems[b % 2])

        def idx_wait(w, b):
            pltpu.make_async_copy(row_hbm.at[pl.ds(base + w * WIN, WIN)],
                                  slots[b], isems[b % 2]).wait()

        def sc_start(b):
            pltpu.async_copy(rows_v, acc_sh.at[slots[b]], ssem0 if b % 2 == 0
                             else ssem1, add=True)

        def sc_wait(b):
            pltpu.make_async_copy(rows_v, acc_sh.at[slots[b]],
                                  ssem0 if b % 2 == 0 else ssem1).wait()

        idx_start(0, 0)
        idx_start(1, 1)

        @pl.loop(0, (N_WIN + 3) // 4)
        def _(g):
            for b in range(4):
                w = 4 * g + b

                @pl.when(w < N_WIN)
                def _():
                    idx_wait(w, b)

                    @pl.when(w >= 2)
                    def _():
                        sc_wait((b + 2) % 4)

                    sc_start(b)

                    @pl.when(w + 2 < N_WIN)
                    def _():
                        idx_start(w + 2, (b + 2) % 4)

        sc_wait((N_WIN - 2) % 4)
        sc_wait((N_WIN - 1) % 4)

    @pl.when(is_deg)
    def _():
        deg_loop()

    @pl.when(jnp.logical_not(is_deg))
    def _():
        window_loop()

    plsc.subcore_barrier()
    o0 = c * N_NODES + r0
    pltpu.sync_copy(acc_sh.at[pl.ds(r0, ROWS_PER_S)],
                    out_hbm.at[pl.ds(o0, ROWS_PER_S)])

    @pl.when(s == 0)
    def _():
        pltpu.sync_copy(
            acc_sh.at[pl.ds(NS * ROWS_PER_S, ROWS_TAIL)],
            out_hbm.at[pl.ds(c * N_NODES + NS * ROWS_PER_S, ROWS_TAIL)])


_sc_spmm = pl.kernel(
    _sc_spmm_body,
    out_type=jax.ShapeDtypeStruct((NC * N_NODES, CH), jnp.float32),
    mesh=_MESH,
    compiler_params=pltpu.CompilerParams(needs_layout_passes=False),
    scratch_types=[
        pltpu.VMEM((WIN,), jnp.int32),
        pltpu.VMEM((WIN,), jnp.int32),
        pltpu.VMEM((WIN,), jnp.int32),
        pltpu.VMEM((WIN,), jnp.int32),
        pltpu.VMEM((WIN,), jnp.int32),
        pltpu.VMEM((16,), jnp.int32),
        pltpu.VMEM((WIN, CH), jnp.float32),
        pltpu.SemaphoreType.DMA,
        pltpu.SemaphoreType.DMA,
        pltpu.SemaphoreType.DMA,
        pltpu.SemaphoreType.DMA,
        pltpu.VMEM_SHARED((N_NODES, CH), jnp.float32),
    ],
)


_BLK = 1000
_GRID = N_NODES // _BLK


def _tc_step_body(sa_ref, sb_ref, x_ref, w_ref, b_ref, kb_ref,
                  dis_ref, acc_ref, disn_ref, y_ref, accn_ref):
    k = kb_ref[0, 0]
    mode0 = k == 0.0
    s = sa_ref[...] + sb_ref[...]
    d0 = s[:, 0:1]
    disc = jnp.where(d0 > 0, lax.rsqrt(jnp.maximum(d0, 1.0)), 0.0)
    dis = jnp.where(mode0, disc, dis_ref[:, 0:1])
    disn_ref[...] = jnp.broadcast_to(dis, (_BLK, 16))
    f = jnp.where(k == 1.0, 1.0, 2.0)
    sub = jnp.where(k == 2.0, 1.0, 0.0)
    t = -(f * (dis * s)) - sub * x_ref[...]
    y_ref[...] = jnp.where(mode0, dis * x_ref[...], dis * t)
    lhs = jnp.where(mode0, x_ref[...], t)
    base = jnp.where(mode0, jnp.broadcast_to(b_ref[...], (_BLK, CH)),
                     acc_ref[...])
    accn_ref[...] = base + jnp.dot(lhs, w_ref[...],
                                   preferred_element_type=jnp.float32,
                                   precision=lax.Precision.HIGHEST)


def _rows_spec(width):
    return pl.BlockSpec((_BLK, width), lambda i: (i, 0))


def _full_spec(shape):
    return pl.BlockSpec(shape, lambda i: (0,) * len(shape))


_tc_step = pl.pallas_call(
    _tc_step_body,
    grid=(_GRID,),
    in_specs=[_rows_spec(CH), _rows_spec(CH), _rows_spec(CH),
              _full_spec((CH, CH)), _full_spec((1, CH)), _full_spec((1, CH)),
              _rows_spec(16), _rows_spec(CH)],
    out_specs=[_rows_spec(16), _rows_spec(CH), _rows_spec(CH)],
    out_shape=[
        jax.ShapeDtypeStruct((N_NODES, 16), jnp.float32),
        jax.ShapeDtypeStruct((N_NODES, CH), jnp.float32),
        jax.ShapeDtypeStruct((N_NODES, CH), jnp.float32),
    ],
)


def kernel(x, edge_index, weight, bias):
    row = edge_index[0]
    col = edge_index[1]
    zeros128 = jnp.zeros((N_NODES, CH), jnp.float32)
    ones_win = jnp.ones((WIN, CH), jnp.float32)
    seq_win = jnp.arange(WIN, dtype=jnp.int32)
    bias2d = bias.reshape(1, CH)

    limit = lax.optimization_barrier(jnp.int32(3))

    def cond(carry):
        k = carry[0]
        return k < limit

    def body(carry):
        k, y, dis16, acc = carry
        flag16 = jnp.full((16,), 1, jnp.int32) * (k == 0).astype(jnp.int32)
        w_k = lax.dynamic_index_in_dim(weight, k, 0, keepdims=False)
        kb = jnp.full((1, CH), 1.0, jnp.float32) * k.astype(jnp.float32)
        sp = _sc_spmm(y, row, col, ones_win, seq_win, zeros128, flag16)
        dis16n, y_next, acc_next = _tc_step(
            sp[:N_NODES], sp[N_NODES:], x, w_k, bias2d, kb, dis16, acc)
        return (k + 1, y_next, dis16n, acc_next)

    carry0 = (jnp.int32(0), x, jnp.zeros((N_NODES, 16), jnp.float32),
              jnp.zeros((N_NODES, CH), jnp.float32))
    _, _, _, out = lax.while_loop(cond, body, carry0)
    return out
